# baseline (speedup 1.0000x reference)
import functools

import jax
import jax.numpy as jnp
from jax.experimental import pallas as pl
from jax.experimental.pallas import tpu as pltpu

_BN_EPS = 1e-5


def _pick_tile(n, cands=(2048, 1024, 512, 256, 128, 64, 32, 16, 8)):
    for c in cands:
        if n % c == 0:
            return c
    return n


def _fused_kernel(x_ref, w_ref, gamma_ref, beta_ref, o_ref,
                  y_scr, w_scr, g_scr, cs_scr,
                  *, num_a, tn, num_b, tm, inv_n):
    j = pl.program_id(0)

    @pl.when(j == 0)
    def _init():
        w_scr[...] = w_ref[...].astype(jnp.bfloat16)
        g_scr[...] = jnp.zeros_like(g_scr)
        cs_scr[...] = jnp.zeros_like(cs_scr)

    @pl.when(j < num_a)
    def _compute_pass():
        x = x_ref[...].astype(jnp.bfloat16)
        y = jnp.dot(x, w_scr[...], preferred_element_type=jnp.float32)
        y_scr[pl.ds(j * tn, tn), :] = y.astype(jnp.bfloat16)
        g_scr[...] += jax.lax.dot_general(
            x, x, (((0,), (0,)), ((), ())),
            preferred_element_type=jnp.float32)
        ones = jnp.ones((8, x.shape[0]), jnp.bfloat16)
        cs_scr[...] += jnp.dot(ones, x, preferred_element_type=jnp.float32)

    @pl.when(j >= num_a)
    def _apply_pass():
        sum8 = jnp.dot(cs_scr[...].astype(jnp.bfloat16), w_scr[...],
                       preferred_element_type=jnp.float32)
        sum_c = jnp.sum(sum8, axis=0, keepdims=True) * 0.125
        gw = jnp.dot(g_scr[...].astype(jnp.bfloat16), w_scr[...],
                     preferred_element_type=jnp.float32)
        sumsq_c = jnp.sum(
            w_ref[...] * gw, axis=0, keepdims=True)
        mean = sum_c * inv_n
        var = jnp.maximum(sumsq_c * inv_n - mean * mean, 0.0)
        scale = gamma_ref[...] * jax.lax.rsqrt(var + _BN_EPS)
        shift = beta_ref[...] - mean * scale
        y = y_scr[pl.ds((j - num_a) * tm, tm), :].astype(jnp.float32)
        o_ref[...] = jnp.maximum(y * scale + shift, 0.0)


@jax.jit
def _forward(src, w, gamma, beta):
    B, T, Din = src.shape
    E = w.shape[1]
    N = B * T
    x2d = src.reshape(N, Din)

    tn = _pick_tile(N)
    num_a = N // tn
    tm = _pick_tile(N)
    num_b = N // tm

    out2d = pl.pallas_call(
        functools.partial(_fused_kernel, num_a=num_a, tn=tn,
                          num_b=num_b, tm=tm, inv_n=1.0 / N),
        out_shape=jax.ShapeDtypeStruct((N, E), src.dtype),
        grid=(num_a + num_b,),
        in_specs=[
            pl.BlockSpec((tn, Din), lambda j: (jnp.minimum(j, num_a - 1), 0)),
            pl.BlockSpec((Din, E), lambda j: (0, 0)),
            pl.BlockSpec((1, E), lambda j: (0, 0)),
            pl.BlockSpec((1, E), lambda j: (0, 0)),
        ],
        out_specs=pl.BlockSpec(
            (tm, E), lambda j: (jnp.maximum(j - num_a, 0), 0)),
        scratch_shapes=[
            pltpu.VMEM((N, E), jnp.bfloat16),
            pltpu.VMEM((Din, E), jnp.bfloat16),
            pltpu.VMEM((Din, Din), jnp.float32),
            pltpu.VMEM((8, Din), jnp.float32),
        ],
        compiler_params=pltpu.CompilerParams(
            dimension_semantics=("arbitrary",),
            vmem_limit_bytes=60 * 1024 * 1024,
        ),
    )(x2d, w, gamma, beta)

    return out2d.reshape(B, T, E)


def kernel(src, w, b, gamma, beta):
    del b
    return _forward(src, w, gamma, beta)


# speedup vs baseline: 1.5114x; 1.5114x over previous
import functools

import jax
import jax.numpy as jnp
from jax.experimental import pallas as pl
from jax.experimental.pallas import tpu as pltpu

_BN_EPS = 1e-5


def _pick_tile(n, cands=(2048, 1024, 512, 256, 128, 64, 32, 16, 8)):
    for c in cands:
        if n % c == 0:
            return c
    return n


def _fused_kernel(x_ref, w_ref, gamma_ref, beta_ref, o_ref,
                  y_scr, w_scr, sum_scr, sumsq_scr,
                  *, num_a, tn, num_b, tm, inv_n):
    j = pl.program_id(0)

    @pl.when(j == 0)
    def _init():
        w_scr[...] = w_ref[...].astype(jnp.bfloat16)
        sum_scr[...] = jnp.zeros_like(sum_scr)
        sumsq_scr[...] = jnp.zeros_like(sumsq_scr)

    @pl.when(j < num_a)
    def _compute_pass():
        x = x_ref[...].astype(jnp.bfloat16)
        y = jnp.dot(x, w_scr[...], preferred_element_type=jnp.float32)
        y_scr[pl.ds(j * tn, tn), :] = y.astype(jnp.bfloat16)
        y3 = y.reshape(-1, 8, y.shape[-1])
        sum_scr[...] += jnp.sum(y3, axis=0)
        sumsq_scr[...] += jnp.sum(y3 * y3, axis=0)

    @pl.when(j >= num_a)
    def _apply_pass():
        sum_c = jnp.sum(sum_scr[...], axis=0, keepdims=True)
        sumsq_c = jnp.sum(sumsq_scr[...], axis=0, keepdims=True)
        mean = sum_c * inv_n
        var = jnp.maximum(sumsq_c * inv_n - mean * mean, 0.0)
        scale = gamma_ref[...] * jax.lax.rsqrt(var + _BN_EPS)
        shift = beta_ref[...] - mean * scale
        y = y_scr[pl.ds((j - num_a) * tm, tm), :].astype(jnp.float32)
        o_ref[...] = jnp.maximum(y * scale + shift, 0.0)


@jax.jit
def _forward(src, w, gamma, beta):
    B, T, Din = src.shape
    E = w.shape[1]
    N = B * T
    x2d = src.reshape(N, Din)

    tn = _pick_tile(N, cands=(4096, 2048, 1024, 512, 256, 128, 64, 32, 16, 8))
    num_a = N // tn
    tm = _pick_tile(N)
    num_b = N // tm

    out2d = pl.pallas_call(
        functools.partial(_fused_kernel, num_a=num_a, tn=tn,
                          num_b=num_b, tm=tm, inv_n=1.0 / N),
        out_shape=jax.ShapeDtypeStruct((N, E), src.dtype),
        grid=(num_a + num_b,),
        in_specs=[
            pl.BlockSpec((tn, Din), lambda j: (jnp.minimum(j, num_a - 1), 0)),
            pl.BlockSpec((Din, E), lambda j: (0, 0)),
            pl.BlockSpec((1, E), lambda j: (0, 0)),
            pl.BlockSpec((1, E), lambda j: (0, 0)),
        ],
        out_specs=pl.BlockSpec(
            (tm, E), lambda j: (jnp.maximum(j - num_a, 0), 0)),
        scratch_shapes=[
            pltpu.VMEM((N, E), jnp.bfloat16),
            pltpu.VMEM((Din, E), jnp.bfloat16),
            pltpu.VMEM((8, E), jnp.float32),
            pltpu.VMEM((8, E), jnp.float32),
        ],
        compiler_params=pltpu.CompilerParams(
            dimension_semantics=("arbitrary",),
            vmem_limit_bytes=65 * 1024 * 1024,
        ),
    )(x2d, w, gamma, beta)

    return out2d.reshape(B, T, E)


def kernel(src, w, b, gamma, beta):
    del b
    return _forward(src, w, gamma, beta)
